# baseline (speedup 1.0000x reference)
import functools
import jax
import jax.numpy as jnp
from jax import lax
from jax.experimental import pallas as pl
from jax.experimental.pallas import tpu as pltpu
from jax.experimental.pallas import tpu_sc as plsc

BATCH = 1024
SEQ = 512
D = 512
NTOK = BATCH * SEQ
NC, NS = 2, 16
NW = NC * NS
PER_W = NTOK // NW
CHUNK = 216
SPLIT = 128
NFULL = PER_W // CHUNK
TAIL = PER_W - NFULL * CHUNK
TSPLIT = 128


def _embed_body(tok_hbm, table_hbm, out_hbm, idx_v, buf, ga, gb, o0):
    wid = lax.axis_index("s") * NC + lax.axis_index("c")
    base = wid * PER_W
    pltpu.sync_copy(tok_hbm.at[wid], idx_v)

    def gathers(c, n, na):
        a = pltpu.make_async_copy(
            table_hbm.at[idx_v.at[pl.ds(c * CHUNK, na)]],
            buf.at[pl.ds(0, na)], ga)
        b = pltpu.make_async_copy(
            table_hbm.at[idx_v.at[pl.ds(c * CHUNK + na, n - na)]],
            buf.at[pl.ds(na, n - na)], gb)
        return a, b

    def out_copy(c, n):
        return pltpu.make_async_copy(
            buf.at[pl.ds(0, n)],
            out_hbm.at[pl.ds(base + c * CHUNK, n)], o0)

    a, b = gathers(0, CHUNK, SPLIT)
    a.start()
    b.start()

    @pl.loop(0, NFULL - 1)
    def _(c):
        a, b = gathers(c, CHUNK, SPLIT)
        a.wait()
        b.wait()
        out_copy(c, CHUNK).start()
        out_copy(c, CHUNK).wait()
        a2, b2 = gathers(c + 1, CHUNK, SPLIT)
        a2.start()
        b2.start()

    c = NFULL - 1
    a, b = gathers(c, CHUNK, SPLIT)
    a.wait()
    b.wait()
    out_copy(c, CHUNK).start()
    out_copy(c, CHUNK).wait()
    a, b = gathers(NFULL, TAIL, TSPLIT)
    a.start()
    b.start()
    a.wait()
    b.wait()
    out_copy(NFULL, TAIL).start()
    out_copy(NFULL, TAIL).wait()


@jax.jit
def _embed(tok, table):
    run = functools.partial(
        pl.kernel,
        out_type=jax.ShapeDtypeStruct((NTOK, D), jnp.float32),
        mesh=plsc.VectorSubcoreMesh(
            core_axis_name="c", subcore_axis_name="s",
            num_cores=NC, num_subcores=NS),
        scratch_types=[
            pltpu.VMEM((PER_W,), jnp.int32),
            pltpu.VMEM((CHUNK, D), jnp.float32),
            pltpu.SemaphoreType.DMA,
            pltpu.SemaphoreType.DMA,
            pltpu.SemaphoreType.DMA,
        ],
    )(_embed_body)
    return run(tok, table)


def kernel(tokens, table):
    tok = tokens.astype(jnp.int32).reshape(NW, PER_W)
    out = _embed(tok, table)
    return out.reshape(BATCH, SEQ, D)


# speedup vs baseline: 1.0138x; 1.0138x over previous
import functools
import jax
import jax.numpy as jnp
from jax import lax
from jax.experimental import pallas as pl
from jax.experimental.pallas import tpu as pltpu
from jax.experimental.pallas import tpu_sc as plsc

BATCH = 1024
SEQ = 512
D = 512
NTOK = BATCH * SEQ
NC, NS = 2, 16
NW = NC * NS
PER_W = NTOK // NW
CHUNK = 104
NFULL = 156
TAILBASE = NFULL * CHUNK
HTAIL = (PER_W - TAILBASE) // 2


def _embed_body(tok_hbm, table_hbm, out_hbm, idx_v, buf0, buf1, g0, g1, o0, o1):
    wid = lax.axis_index("s") * NC + lax.axis_index("c")
    base = wid * PER_W
    pltpu.sync_copy(tok_hbm.at[wid], idx_v)

    bufs = (buf0, buf1)
    gsems = (g0, g1)
    osems = (o0, o1)

    def gather(c, b):
        return pltpu.make_async_copy(
            table_hbm.at[idx_v.at[pl.ds(c * CHUNK, CHUNK)]], bufs[b], gsems[b])

    def out_copy(c, b):
        return pltpu.make_async_copy(
            bufs[b], out_hbm.at[pl.ds(base + c * CHUNK, CHUNK)], osems[b])

    gather(0, 0).start()
    gather(0, 0).wait()
    out_copy(0, 0).start()
    gather(1, 1).start()
    gather(1, 1).wait()
    out_copy(1, 1).start()
    out_copy(0, 0).wait()
    gather(2, 0).start()

    @pl.loop(1, NFULL // 2 - 1)
    def _(p):
        c = 2 * p
        gather(c, 0).wait()
        out_copy(c, 0).start()
        out_copy(c - 1, 1).wait()
        gather(c + 1, 1).start()
        gather(c + 1, 1).wait()
        out_copy(c + 1, 1).start()
        out_copy(c, 0).wait()
        gather(c + 2, 0).start()

    c = NFULL - 2
    gather(c, 0).wait()
    out_copy(c, 0).start()
    out_copy(c - 1, 1).wait()
    gather(c + 1, 1).start()
    gather(c + 1, 1).wait()
    out_copy(c + 1, 1).start()
    out_copy(c, 0).wait()

    def tail_g(h, b):
        return pltpu.make_async_copy(
            table_hbm.at[idx_v.at[pl.ds(TAILBASE + h * HTAIL, HTAIL)]],
            bufs[b].at[pl.ds(0, HTAIL)], gsems[b])

    def tail_o(h, b):
        return pltpu.make_async_copy(
            bufs[b].at[pl.ds(0, HTAIL)],
            out_hbm.at[pl.ds(base + TAILBASE + h * HTAIL, HTAIL)], osems[b])

    tail_g(0, 0).start()
    out_copy(c + 1, 1).wait()
    tail_g(1, 1).start()
    tail_g(0, 0).wait()
    tail_o(0, 0).start()
    tail_g(1, 1).wait()
    tail_o(1, 1).start()
    tail_o(0, 0).wait()
    tail_o(1, 1).wait()


@jax.jit
def _embed(tok, table):
    run = functools.partial(
        pl.kernel,
        out_type=jax.ShapeDtypeStruct((NTOK, D), jnp.float32),
        mesh=plsc.VectorSubcoreMesh(
            core_axis_name="c", subcore_axis_name="s",
            num_cores=NC, num_subcores=NS),
        scratch_types=[
            pltpu.VMEM((PER_W,), jnp.int32),
            pltpu.VMEM((CHUNK, D), jnp.float32),
            pltpu.VMEM((CHUNK, D), jnp.float32),
            pltpu.SemaphoreType.DMA,
            pltpu.SemaphoreType.DMA,
            pltpu.SemaphoreType.DMA,
            pltpu.SemaphoreType.DMA,
        ],
    )(_embed_body)
    return run(tok, table)


def kernel(tokens, table):
    tok = tokens.astype(jnp.int32).reshape(NW, PER_W)
    out = _embed(tok, table)
    return out.reshape(BATCH, SEQ, D)


# speedup vs baseline: 1.0156x; 1.0018x over previous
import functools
import jax
import jax.numpy as jnp
from jax import lax
from jax.experimental import pallas as pl
from jax.experimental.pallas import tpu as pltpu
from jax.experimental.pallas import tpu_sc as plsc

BATCH = 1024
SEQ = 512
D = 512
NTOK = BATCH * SEQ
NC, NS = 2, 16
NW = NC * NS
PER_W = NTOK // NW
CHUNK = 96
NFULL = 170
TAIL = PER_W - NFULL * CHUNK


def _embed_body(tok_hbm, table_hbm, out_hbm, idx_v, buf0, buf1, g0, g1, o0, o1):
    wid = lax.axis_index("s") * NC + lax.axis_index("c")
    base = wid * PER_W
    pltpu.sync_copy(tok_hbm.at[wid], idx_v)

    bufs = (buf0, buf1)
    gsems = (g0, g1)
    osems = (o0, o1)

    def gather(c, b):
        return pltpu.make_async_copy(
            table_hbm.at[idx_v.at[pl.ds(c * CHUNK, CHUNK)]], bufs[b], gsems[b])

    def out_copy(c, b):
        return pltpu.make_async_copy(
            bufs[b], out_hbm.at[pl.ds(base + c * CHUNK, CHUNK)], osems[b])

    gather(0, 0).start()
    gather(0, 0).wait()
    out_copy(0, 0).start()
    gather(1, 1).start()
    gather(1, 1).wait()
    out_copy(1, 1).start()
    out_copy(0, 0).wait()
    gather(2, 0).start()

    @pl.loop(1, NFULL // 2 - 1)
    def _(p):
        c = 2 * p
        gather(c, 0).wait()
        out_copy(c, 0).start()
        out_copy(c - 1, 1).wait()
        gather(c + 1, 1).start()
        gather(c + 1, 1).wait()
        out_copy(c + 1, 1).start()
        out_copy(c, 0).wait()
        gather(c + 2, 0).start()

    c = NFULL - 2
    gather(c, 0).wait()
    out_copy(c, 0).start()
    out_copy(c - 1, 1).wait()
    gather(c + 1, 1).start()
    gather(c + 1, 1).wait()
    out_copy(c + 1, 1).start()
    out_copy(c, 0).wait()
    tail_g = pltpu.make_async_copy(
        table_hbm.at[idx_v.at[pl.ds(NFULL * CHUNK, TAIL)]],
        buf0.at[pl.ds(0, TAIL)], gsems[0])
    tail_o = pltpu.make_async_copy(
        buf0.at[pl.ds(0, TAIL)],
        out_hbm.at[pl.ds(base + NFULL * CHUNK, TAIL)], osems[0])
    tail_g.start()
    tail_g.wait()
    tail_o.start()
    out_copy(c + 1, 1).wait()
    tail_o.wait()


@jax.jit
def _embed(tok, table):
    run = functools.partial(
        pl.kernel,
        out_type=jax.ShapeDtypeStruct((NTOK, D), jnp.float32),
        mesh=plsc.VectorSubcoreMesh(
            core_axis_name="c", subcore_axis_name="s",
            num_cores=NC, num_subcores=NS),
        scratch_types=[
            pltpu.VMEM((PER_W,), jnp.int32),
            pltpu.VMEM((CHUNK, D), jnp.float32),
            pltpu.VMEM((CHUNK, D), jnp.float32),
            pltpu.SemaphoreType.DMA,
            pltpu.SemaphoreType.DMA,
            pltpu.SemaphoreType.DMA,
            pltpu.SemaphoreType.DMA,
        ],
    )(_embed_body)
    return run(tok, table)


def kernel(tokens, table):
    tok = tokens.astype(jnp.int32).reshape(NW, PER_W)
    out = _embed(tok, table)
    return out.reshape(BATCH, SEQ, D)
